# Initial kernel scaffold; baseline (speedup 1.0000x reference)
import jax
import jax.numpy as jnp
from jax.experimental import pallas as pl


def kernel(sensory_features, W1, b1, W2, b2, fear_memory, extinction_memory):
    raise NotImplementedError("write your pallas kernel here")


# speedup vs baseline: 2.1604x; 2.1604x over previous
import functools

import jax
import jax.numpy as jnp
from jax.experimental import pallas as pl

_EPS = 1e-8


def _fear_kernel(x_ref, w1_ref, b1_ref, w2_ref, b2_ref, f_ref, e_ref, o_ref):
    x = x_ref[...]

    h = jax.lax.dot_general(
        x, w1_ref[...], (((1,), (1,)), ((), ())),
        preferred_element_type=jnp.float32)
    h = jnp.maximum(h + b1_ref[...], 0.0)
    logits = jax.lax.dot_general(
        h, w2_ref[...], (((1,), (1,)), ((), ())),
        preferred_element_type=jnp.float32)
    logits = logits + b2_ref[...]
    m = jnp.max(logits, axis=-1, keepdims=True)
    ex = jnp.exp(logits - m)
    context = ex / jnp.sum(ex, axis=-1, keepdims=True)

    x_norm = jnp.maximum(jnp.sqrt(jnp.sum(x * x, axis=-1)), _EPS)

    def cos_recall(mem):
        assoc = jnp.dot(context, mem, preferred_element_type=jnp.float32)
        dot = jnp.sum(x * assoc, axis=-1)
        a_norm = jnp.maximum(jnp.sqrt(jnp.sum(assoc * assoc, axis=-1)), _EPS)
        return dot / (x_norm * a_norm)

    sim = cos_recall(f_ref[...]) - cos_recall(e_ref[...])
    o_ref[...] = jax.nn.sigmoid(sim)[:, None]


@functools.partial(jax.jit, static_argnames=())
def kernel(sensory_features, W1, b1, W2, b2, fear_memory, extinction_memory):
    B, D = sensory_features.shape
    H = W1.shape[0]
    C = W2.shape[0]
    BB = 1024

    rep = lambda i: (0, 0)
    out = pl.pallas_call(
        _fear_kernel,
        grid=(B // BB,),
        in_specs=[
            pl.BlockSpec((BB, D), lambda i: (i, 0)),
            pl.BlockSpec((H, D), rep),
            pl.BlockSpec((1, H), rep),
            pl.BlockSpec((C, H), rep),
            pl.BlockSpec((1, C), rep),
            pl.BlockSpec((C, D), rep),
            pl.BlockSpec((C, D), rep),
        ],
        out_specs=pl.BlockSpec((BB, 1), lambda i: (i, 0)),
        out_shape=jax.ShapeDtypeStruct((B, 1), jnp.float32),
    )(sensory_features, W1, b1.reshape(1, H), W2, b2.reshape(1, C),
      fear_memory, extinction_memory)
    return out
